# Initial kernel scaffold; baseline (speedup 1.0000x reference)
import jax
import jax.numpy as jnp
from jax.experimental import pallas as pl


def kernel(queries, Wq, bq, keys, w_down_embed, w_up_embed):
    raise NotImplementedError("write your pallas kernel here")


# speedup vs baseline: 3.7597x; 3.7597x over previous
import functools

import jax
import jax.numpy as jnp
from jax import lax
from jax.experimental import pallas as pl
from jax.experimental.pallas import tpu as pltpu
from jax.experimental.pallas import tpu_sc as plsc

D_MODEL = 1024
N_HEADS = 8
D_KEYS = 128
HALF = D_KEYS // 2
N_EXPERTS = 256
K = 8
NTOK = 256


def _topk(s, k, payload=None):
    r, c = s.shape
    col = lax.broadcasted_iota(jnp.int32, (r, c), 1)
    vals, idxs, pays = [], [], []
    cur = s
    for _ in range(k):
        m = jnp.max(cur, axis=1, keepdims=True)
        hit = cur == m
        idx = jnp.min(jnp.where(hit, col, c), axis=1, keepdims=True)
        sel = col == idx
        vals.append(m)
        idxs.append(idx)
        if payload is not None:
            pays.append(jnp.sum(jnp.where(sel, payload, 0), axis=1, keepdims=True))
        cur = jnp.where(sel, -jnp.inf, cur)
    out_v = jnp.concatenate(vals, axis=1)
    out_i = jnp.concatenate(idxs, axis=1)
    if payload is not None:
        return out_v, out_i, jnp.concatenate(pays, axis=1)
    return out_v, out_i


def _routing_body(x_ref, wq_ref, bq_ref, keys_ref, gate_ref, idx_ref):
    x = x_ref[...]
    q = lax.dot_general(x, wq_ref[...], (((1,), (1,)), ((), ())),
                        preferred_element_type=jnp.float32)
    q = q + bq_ref[...][None, :]
    for h in range(N_HEADS):
        q1 = q[:, h * D_KEYS: h * D_KEYS + HALF]
        q2 = q[:, h * D_KEYS + HALF: (h + 1) * D_KEYS]
        k1 = keys_ref[h, 0]
        k2 = keys_ref[h, 1]
        s1 = lax.dot_general(q1, k1, (((1,), (1,)), ((), ())),
                             preferred_element_type=jnp.float32)
        s2 = lax.dot_general(q2, k2, (((1,), (1,)), ((), ())),
                             preferred_element_type=jnp.float32)
        sv1, iv1 = _topk(s1, K)
        sv2, iv2 = _topk(s2, K)
        all_s = jnp.concatenate([sv1[:, i:i + 1] + sv2 for i in range(K)], axis=1)
        all_i = jnp.concatenate([iv1[:, i:i + 1] * N_EXPERTS + iv2 for i in range(K)],
                                axis=1)
        sc, _, sel_i = _topk(all_s, K, payload=all_i)
        m = jnp.max(sc, axis=1, keepdims=True)
        e = jnp.exp(sc - m)
        g = e / jnp.sum(e, axis=1, keepdims=True)
        gate_ref[h] = g
        idx_ref[h] = sel_i


def _routing(x, Wq, bq, keys):
    return pl.pallas_call(
        _routing_body,
        out_shape=[
            jax.ShapeDtypeStruct((N_HEADS, NTOK, K), jnp.float32),
            jax.ShapeDtypeStruct((N_HEADS, NTOK, K), jnp.int32),
        ],
    )(x, Wq, bq, keys)


_NW = 32
_ROWS = N_HEADS * NTOK * K
_PER_W = _ROWS // _NW
_CHUNK = 64
_NCH = _PER_W // _CHUNK


def _gather_kernel_body(idx_hbm, down_hbm, up_hbm, out_down, out_up,
                        idx_v, rows_v, sem):
    wid = lax.axis_index("s") * 2 + lax.axis_index("c")
    base = wid * _PER_W
    for t, (table, out) in enumerate(((down_hbm, out_down), (up_hbm, out_up))):
        for i in range(_NCH):
            off = base + i * _CHUNK
            pltpu.sync_copy(idx_hbm.at[pl.ds(off, _CHUNK)], idx_v)
            pltpu.async_copy(table.at[idx_v], rows_v, sem).wait()
            pltpu.sync_copy(rows_v, out.at[pl.ds(off, _CHUNK)])


@functools.cache
def _make_gather():
    return functools.partial(
        pl.kernel,
        mesh=plsc.VectorSubcoreMesh(core_axis_name="c", subcore_axis_name="s"),
        out_type=[
            jax.ShapeDtypeStruct((_ROWS, D_MODEL), jnp.float32),
            jax.ShapeDtypeStruct((_ROWS, D_MODEL), jnp.float32),
        ],
        scratch_types=[
            pltpu.VMEM((_CHUNK,), jnp.int32),
            pltpu.VMEM((_CHUNK, D_MODEL), jnp.float32),
            pltpu.SemaphoreType.DMA,
        ],
    )(_gather_kernel_body)


def _gather(idx_flat, down, up):
    return _make_gather()(idx_flat, down, up)


_JBLK = 256
_NJ = _ROWS // _JBLK


def _ffn_body(x_ref, wd_ref, wu_ref, g_ref, o_ref, acc_ref):
    j = pl.program_id(0)

    @pl.when(j == 0)
    def _():
        acc_ref[...] = jnp.zeros_like(acc_ref)

    xc = lax.dot_general(x_ref[...], wd_ref[...], (((1,), (1,)), ((), ())),
                         preferred_element_type=jnp.float32)
    xc = jnp.maximum(xc, 0.0) * g_ref[0]
    acc_ref[...] += lax.dot_general(xc, wu_ref[...], (((1,), (0,)), ((), ())),
                                    preferred_element_type=jnp.float32)

    @pl.when(j == _NJ - 1)
    def _():
        o_ref[...] = acc_ref[...]


def _ffn(x, wd, wu, gate3):
    return pl.pallas_call(
        _ffn_body,
        grid=(_NJ,),
        in_specs=[
            pl.BlockSpec((NTOK, D_MODEL), lambda j: (0, 0)),
            pl.BlockSpec((_JBLK, D_MODEL), lambda j: (j, 0)),
            pl.BlockSpec((_JBLK, D_MODEL), lambda j: (j, 0)),
            pl.BlockSpec((1, 1, _JBLK), lambda j: (j, 0, 0)),
        ],
        out_specs=pl.BlockSpec((NTOK, D_MODEL), lambda j: (0, 0)),
        out_shape=jax.ShapeDtypeStruct((NTOK, D_MODEL), jnp.float32),
        scratch_shapes=[pltpu.VMEM((NTOK, D_MODEL), jnp.float32)],
    )(x, wd, wu, gate3)


def kernel(queries, Wq, bq, keys, w_down_embed, w_up_embed):
    n, t, d = queries.shape
    x = queries.reshape(-1, d)
    gate, idx = _routing(x, Wq, bq, keys)
    idx_flat = idx.reshape(-1)
    wd, wu = _gather(idx_flat, w_down_embed, w_up_embed)
    gate3 = gate.reshape(_NJ, 1, _JBLK)
    out = _ffn(x, wd, wu, gate3)
    return out.reshape(n, t, d)
